# baseline (speedup 1.0000x reference)
import functools

import jax
import jax.numpy as jnp
from jax import lax
from jax.experimental import pallas as pl
from jax.experimental.pallas import tpu as pltpu
from jax.experimental.pallas import tpu_sc as plsc

_VOCAB = 1000
_C = 64
_CP = 128
_T = 8
_B = 4096

_NC = 2
_NS = 16
_NW = _NC * _NS
_ROWS = _B * _T
_RPW = _ROWS // _NW
_CHUNK = 128
_NCHUNK = _RPW // _CHUNK

_RB = 128


def _sc_gather(tok_pad, idx2):
    mesh = plsc.VectorSubcoreMesh(core_axis_name="c", subcore_axis_name="s")

    @functools.partial(
        pl.kernel,
        mesh=mesh,
        out_type=jax.ShapeDtypeStruct((_ROWS, _CP), jnp.float32),
        scratch_types=[
            pltpu.VMEM((_NCHUNK, _CHUNK), jnp.int32),
            pltpu.VMEM((2, _CHUNK, _CP), jnp.float32),
            pltpu.SemaphoreType.DMA,
            pltpu.SemaphoreType.DMA,
        ],
    )
    def k(tok_hbm, idx_hbm, out_hbm, idx_v, buf, sem0, sem1):
        wid = lax.axis_index("s") * _NC + lax.axis_index("c")
        base = wid * _NCHUNK
        pltpu.sync_copy(idx_hbm.at[pl.ds(base, _NCHUNK)], idx_v)
        sems = [sem0, sem1]
        copies = [None, None]
        copies[0] = pltpu.async_copy(
            tok_hbm.at[idx_v.at[0]], buf.at[0], sems[0])
        for j in range(_NCHUNK):
            if j + 1 < _NCHUNK:
                copies[(j + 1) % 2] = pltpu.async_copy(
                    tok_hbm.at[idx_v.at[j + 1]], buf.at[(j + 1) % 2],
                    sems[(j + 1) % 2])
            copies[j % 2].wait()
            pltpu.sync_copy(buf.at[j % 2],
                            out_hbm.at[pl.ds((base + j) * _CHUNK, _CHUNK)])

    return k(tok_pad, idx2)


def _tc_body(x_ref, pos_ref, w_ref, b_ref, o_ref):
    x = x_ref[...] + pos_ref[...][None, :, :]
    y = jnp.dot(x.reshape(_RB * _T, _CP), w_ref[...],
                preferred_element_type=jnp.float32) + b_ref[...]
    o_ref[...] = y.reshape(_RB, _T, _VOCAB)


def kernel(idx, tok_table, pos_table, W, b):
    B, T = idx.shape
    tok_pad = jnp.pad(tok_table, ((0, 0), (0, _CP - _C)))
    pos_pad = jnp.pad(pos_table, ((0, 0), (0, _CP - _C)))
    W_pad = jnp.pad(W, ((0, _CP - _C), (0, 0)))
    idx2 = idx.reshape(_NW * _NCHUNK, _CHUNK)
    tok_emb = _sc_gather(tok_pad, idx2).reshape(B, T, _CP)

    out = pl.pallas_call(
        _tc_body,
        grid=(B // _RB,),
        in_specs=[
            pl.BlockSpec((_RB, _T, _CP), lambda i: (i, 0, 0)),
            pl.BlockSpec((_T, _CP), lambda i: (0, 0)),
            pl.BlockSpec((_CP, _VOCAB), lambda i: (0, 0)),
            pl.BlockSpec((1, _VOCAB), lambda i: (0, 0)),
        ],
        out_specs=pl.BlockSpec((_RB, _T, _VOCAB), lambda i: (i, 0, 0)),
        out_shape=jax.ShapeDtypeStruct((B, T, _VOCAB), jnp.float32),
    )(tok_emb, pos_pad, W_pad, b.reshape(1, _VOCAB))

    return out


# speedup vs baseline: 1.0280x; 1.0280x over previous
import functools

import jax
import jax.numpy as jnp
from jax import lax
from jax.experimental import pallas as pl
from jax.experimental.pallas import tpu as pltpu
from jax.experimental.pallas import tpu_sc as plsc

_VOCAB = 1000
_C = 64
_CP = 128
_T = 8
_B = 4096

_NC = 2
_NS = 16
_NW = _NC * _NS
_ROWS = _B * _T
_RPW = _ROWS // _NW
_CHUNK = 128
_NCHUNK = _RPW // _CHUNK

_RB = 128
_NBUF = 4


def _sc_gather(tok_pad, idx2):
    mesh = plsc.VectorSubcoreMesh(core_axis_name="c", subcore_axis_name="s")

    @functools.partial(
        pl.kernel,
        mesh=mesh,
        out_type=jax.ShapeDtypeStruct((_ROWS, _CP), jnp.float32),
        scratch_types=[
            pltpu.VMEM((_NCHUNK, _CHUNK), jnp.int32),
            pltpu.VMEM((2, _CHUNK, _CP), jnp.float32),
            pltpu.SemaphoreType.DMA,
            pltpu.SemaphoreType.DMA,
        ],
    )
    def k(tok_hbm, idx_hbm, out_hbm, idx_v, buf, sem0, sem1):
        wid = lax.axis_index("s") * _NC + lax.axis_index("c")
        base = wid * _NCHUNK
        pltpu.sync_copy(idx_hbm.at[pl.ds(base, _NCHUNK)], idx_v)
        sems = [sem0, sem1]
        copies = [None, None]
        copies[0] = pltpu.async_copy(
            tok_hbm.at[idx_v.at[0]], buf.at[0], sems[0])
        for j in range(_NCHUNK):
            if j + 1 < _NCHUNK:
                copies[(j + 1) % 2] = pltpu.async_copy(
                    tok_hbm.at[idx_v.at[j + 1]], buf.at[(j + 1) % 2],
                    sems[(j + 1) % 2])
            copies[j % 2].wait()
            pltpu.sync_copy(buf.at[j % 2],
                            out_hbm.at[pl.ds((base + j) * _CHUNK, _CHUNK)])

    return k(tok_pad, idx2)


def _tc_body(x_ref, pos_ref, w_ref, b_ref, o_hbm, acc, sem):
    i = pl.program_id(0)
    nblk = pl.num_programs(0)
    slot = lax.rem(i, _NBUF)

    def out_copy(j):
        return pltpu.make_async_copy(
            acc.at[lax.rem(j, _NBUF)],
            o_hbm.at[pl.ds(j * _RB, _RB)],
            sem.at[lax.rem(j, _NBUF)])

    @pl.when(i >= _NBUF)
    def _():
        out_copy(i - _NBUF).wait()

    x = x_ref[...].reshape(_RB, _T, _CP) + pos_ref[...][None, :, :]
    y = jnp.dot(x.reshape(_RB * _T, _CP), w_ref[...],
                preferred_element_type=jnp.float32) + b_ref[...]
    acc[slot] = y.reshape(_RB, _T, _VOCAB)
    out_copy(i).start()

    @pl.when(i == nblk - 1)
    def _():
        for k in range(_NBUF):
            out_copy(nblk - _NBUF + k).wait()


def kernel(idx, tok_table, pos_table, W, b):
    B, T = idx.shape
    tok_pad = jnp.pad(tok_table, ((0, 0), (0, _CP - _C)))
    pos_pad = jnp.pad(pos_table, ((0, 0), (0, _CP - _C)))
    W_pad = jnp.pad(W, ((0, _CP - _C), (0, 0)))
    idx2 = idx.reshape(_NW * _NCHUNK, _CHUNK)
    tok_emb = _sc_gather(tok_pad, idx2)

    out = pl.pallas_call(
        _tc_body,
        grid=(B // _RB,),
        in_specs=[
            pl.BlockSpec((_RB * _T, _CP), lambda i: (i, 0)),
            pl.BlockSpec((_T, _CP), lambda i: (0, 0)),
            pl.BlockSpec((_CP, _VOCAB), lambda i: (0, 0)),
            pl.BlockSpec((1, _VOCAB), lambda i: (0, 0)),
        ],
        out_specs=pl.BlockSpec(memory_space=pltpu.MemorySpace.HBM),
        out_shape=jax.ShapeDtypeStruct((B, T, _VOCAB), jnp.float32),
        scratch_shapes=[
            pltpu.VMEM((_NBUF, _RB, _T, _VOCAB), jnp.float32),
            pltpu.SemaphoreType.DMA((_NBUF,)),
        ],
    )(tok_emb, pos_pad, W_pad, b.reshape(1, _VOCAB))

    return out
